# baseline (speedup 1.0000x reference)
import functools
import math

import jax
import jax.numpy as jnp
from jax import lax
from jax.experimental import pallas as pl
from jax.experimental.pallas import tpu as pltpu
from jax.experimental.pallas import tpu_sc as plsc

N_OBS = 2097152
OBS_DIM = 3
BINS = 256
COEF = math.sqrt(2.0 * math.log(100000.0))

NC = 2
NS = 16
L = 16
NW = NC * NS
B_PER_W = N_OBS // NW
CH = 2048
N_CH = B_PER_W // CH
G = CH // L
ROWS = CH // 128


def _ucb_sc_kernel(obs_hbm, table_hbm, out_hbm, obs_v, idx_v, vals_v,
                   out_v, sem):
    wid = lax.axis_index("s") * NC + lax.axis_index("c")
    base = wid * B_PER_W

    lane = lax.broadcasted_iota(jnp.int32, (L,), 0)

    def chunk_body(c, _):
        start = base + c * CH
        pltpu.sync_copy(obs_hbm.at[pl.ds(start * 3, CH * 3)], obs_v)

        def group_body(g, _):
            ii = 48 * g + 3 * lane
            x0 = plsc.load_gather(obs_v, [ii])
            x1 = plsc.load_gather(obs_v, [ii + 1])
            x2 = plsc.load_gather(obs_v, [ii + 2])
            f0 = (x0 * float(BINS)).astype(jnp.int32)
            f1 = (x1 * float(BINS)).astype(jnp.int32)
            f2 = (x2 * float(BINS)).astype(jnp.int32)
            flat = ((f0 << 8) | f1) << 8 | f2
            r = g // 8
            off = (g % 8) * L
            idx_v[r, pl.ds(off, L)] = flat
            return 0

        lax.fori_loop(0, G, group_body, 0, unroll=4)

        copies = [
            pltpu.make_async_copy(table_hbm.at[idx_v.at[r]], vals_v.at[r],
                                  sem)
            for r in range(ROWS)
        ]
        for cp in copies:
            cp.start()
        for cp in copies:
            cp.wait()

        def bonus_body(g, _):
            r = g // 8
            off = (g % 8) * L
            n = vals_v[r, pl.ds(off, L)]
            i = plsc.bitcast(n, jnp.int32)
            y = plsc.bitcast(jnp.int32(0x5F3759DF) - (i >> 1), jnp.float32)
            hn = 0.5 * n
            y = y * (1.5 - hn * y * y)
            y = y * (1.5 - hn * y * y)
            out_v[pl.ds(g * L, L)] = COEF * y
            return 0

        lax.fori_loop(0, G, bonus_body, 0, unroll=4)

        pltpu.sync_copy(out_v, out_hbm.at[pl.ds(start, CH)])
        return 0

    lax.fori_loop(0, N_CH, chunk_body, 0)


@jax.jit
def kernel(ob_no, state_counts):
    obs_flat = ob_no.reshape(-1)
    table_flat = state_counts.reshape(-1)
    mesh = plsc.VectorSubcoreMesh(core_axis_name="c", subcore_axis_name="s",
                                  num_cores=NC, num_subcores=NS)
    run = pl.kernel(
        _ucb_sc_kernel,
        out_type=jax.ShapeDtypeStruct((N_OBS,), jnp.float32),
        mesh=mesh,
        scratch_types=[
            pltpu.VMEM((CH * 3,), jnp.float32),
            pltpu.VMEM((ROWS, 128), jnp.int32),
            pltpu.VMEM((ROWS, 128), jnp.float32),
            pltpu.VMEM((CH,), jnp.float32),
            pltpu.SemaphoreType.DMA,
        ],
        compiler_params=pltpu.CompilerParams(needs_layout_passes=False),
    )
    return run(obs_flat, table_flat)


# speedup vs baseline: 1.0005x; 1.0005x over previous
import functools
import math

import jax
import jax.numpy as jnp
from jax import lax
from jax.experimental import pallas as pl
from jax.experimental.pallas import tpu as pltpu
from jax.experimental.pallas import tpu_sc as plsc

N_OBS = 2097152
OBS_DIM = 3
BINS = 256
COEF = math.sqrt(2.0 * math.log(100000.0))

NC = 2
NS = 16
L = 16
NW = NC * NS
B_PER_W = N_OBS // NW
CH = 2048
N_CH = B_PER_W // CH
G = CH // L
ROWS = CH // 128


def _ucb_sc_kernel(obs_hbm, table_hbm, out_hbm, obs_v, idx_v, vals_v,
                   out_v, sem):
    wid = lax.axis_index("s") * NC + lax.axis_index("c")
    base = wid * B_PER_W

    lane = lax.broadcasted_iota(jnp.int32, (L,), 0)

    def chunk_body(c, _):
        start = base + c * CH
        pltpu.sync_copy(obs_hbm.at[pl.ds(start * 3, CH * 3)], obs_v)

        def group_body(g, _):
            ii = 48 * g + 3 * lane
            x0 = plsc.load_gather(obs_v, [ii])
            x1 = plsc.load_gather(obs_v, [ii + 1])
            x2 = plsc.load_gather(obs_v, [ii + 2])
            f0 = (x0 * float(BINS)).astype(jnp.int32)
            f1 = (x1 * float(BINS)).astype(jnp.int32)
            f2 = (x2 * float(BINS)).astype(jnp.int32)
            flat = ((f0 << 8) | f1) << 8 | f2
            r = g // 8
            off = (g % 8) * L
            idx_v[r, pl.ds(off, L)] = flat
            return 0

        lax.fori_loop(0, G, group_body, 0, unroll=4)

        copies = [
            pltpu.make_async_copy(table_hbm.at[idx_v.at[r]], vals_v.at[r],
                                  sem)
            for r in range(ROWS)
        ]
        for cp in copies:
            cp.start()
        for cp in copies:
            cp.wait()

        def bonus_body(g, _):
            r = g // 8
            off = (g % 8) * L
            n = vals_v[r, pl.ds(off, L)]
            i = plsc.bitcast(n, jnp.int32)
            y = plsc.bitcast(jnp.int32(0x5F3759DF) - (i >> 1), jnp.float32)
            hn = 0.5 * n
            y = y * (1.5 - hn * y * y)
            y = y * (1.5 - hn * y * y)
            out_v[pl.ds(g * L, L)] = COEF * y
            return 0

        lax.fori_loop(0, G, bonus_body, 0, unroll=4)

        pltpu.sync_copy(out_v, out_hbm.at[pl.ds(start, CH)])
        return 0

    lax.fori_loop(0, N_CH, chunk_body, 0)


@jax.jit
def kernel(ob_no, state_counts):
    obs_flat = ob_no.reshape(-1)
    table_flat = state_counts.reshape(-1)
    mesh = plsc.VectorSubcoreMesh(core_axis_name="c", subcore_axis_name="s",
                                  num_cores=NC, num_subcores=NS)
    run = pl.kernel(
        _ucb_sc_kernel,
        out_type=jax.ShapeDtypeStruct((N_OBS,), jnp.float32),
        mesh=mesh,
        scratch_types=[
            pltpu.VMEM((CH * OBS_DIM,), jnp.float32),
            pltpu.VMEM((ROWS, 128), jnp.int32),
            pltpu.VMEM((ROWS, 128), jnp.float32),
            pltpu.VMEM((CH,), jnp.float32),
            pltpu.SemaphoreType.DMA,
        ],
        compiler_params=pltpu.CompilerParams(needs_layout_passes=False,
                                             use_tc_tiling_on_sc=False),
    )
    return run(obs_flat, table_flat)


# speedup vs baseline: 4.4594x; 4.4571x over previous
import functools
import math

import jax
import jax.numpy as jnp
from jax import lax
from jax.experimental import pallas as pl
from jax.experimental.pallas import tpu as pltpu
from jax.experimental.pallas import tpu_sc as plsc

N_OBS = 2097152
OBS_DIM = 3
BINS = 256
COEF = math.sqrt(2.0 * math.log(100000.0))

NC = 2
NS = 16
L = 16
NW = NC * NS
B_PER_W = N_OBS // NW
CH = 2048
N_CH = B_PER_W // CH
G = CH // L
ROWS = CH // 128


def _ucb_sc_kernel(obs_hbm, table_hbm, out_hbm, xv, yv, zv, idx_v, vals_v,
                   out_v, sem):
    wid = lax.axis_index("s") * NC + lax.axis_index("c")
    base = wid * B_PER_W

    def chunk_body(c, _):
        start = base + c * CH
        pltpu.sync_copy(obs_hbm.at[0, pl.ds(start, CH)], xv)
        pltpu.sync_copy(obs_hbm.at[1, pl.ds(start, CH)], yv)
        pltpu.sync_copy(obs_hbm.at[2, pl.ds(start, CH)], zv)

        def group_body(g, _):
            s = pl.ds(g * L, L)
            f0 = (xv[s] * float(BINS)).astype(jnp.int32)
            f1 = (yv[s] * float(BINS)).astype(jnp.int32)
            f2 = (zv[s] * float(BINS)).astype(jnp.int32)
            flat = ((f0 << 8) | f1) << 8 | f2
            r = g // 8
            off = (g % 8) * L
            idx_v[r, pl.ds(off, L)] = flat
            return 0

        lax.fori_loop(0, G, group_body, 0, unroll=4)

        copies = [
            pltpu.make_async_copy(table_hbm.at[idx_v.at[r]], vals_v.at[r],
                                  sem)
            for r in range(ROWS)
        ]
        for cp in copies:
            cp.start()
        for cp in copies:
            cp.wait()

        def bonus_body(g, _):
            r = g // 8
            off = (g % 8) * L
            n = vals_v[r, pl.ds(off, L)]
            i = plsc.bitcast(n, jnp.int32)
            y = plsc.bitcast(jnp.int32(0x5F3759DF) - (i >> 1), jnp.float32)
            hn = 0.5 * n
            y = y * (1.5 - hn * y * y)
            y = y * (1.5 - hn * y * y)
            out_v[pl.ds(g * L, L)] = COEF * y
            return 0

        lax.fori_loop(0, G, bonus_body, 0, unroll=4)

        pltpu.sync_copy(out_v, out_hbm.at[pl.ds(start, CH)])
        return 0

    lax.fori_loop(0, N_CH, chunk_body, 0)


@jax.jit
def kernel(ob_no, state_counts):
    obs_t = ob_no.T
    table_flat = state_counts.reshape(-1)
    mesh = plsc.VectorSubcoreMesh(core_axis_name="c", subcore_axis_name="s",
                                  num_cores=NC, num_subcores=NS)
    run = pl.kernel(
        _ucb_sc_kernel,
        out_type=jax.ShapeDtypeStruct((N_OBS,), jnp.float32),
        mesh=mesh,
        scratch_types=[
            pltpu.VMEM((CH,), jnp.float32),
            pltpu.VMEM((CH,), jnp.float32),
            pltpu.VMEM((CH,), jnp.float32),
            pltpu.VMEM((ROWS, 128), jnp.int32),
            pltpu.VMEM((ROWS, 128), jnp.float32),
            pltpu.VMEM((CH,), jnp.float32),
            pltpu.SemaphoreType.DMA,
        ],
        compiler_params=pltpu.CompilerParams(needs_layout_passes=False,
                                             use_tc_tiling_on_sc=False),
    )
    return run(obs_t, table_flat)


# speedup vs baseline: 9.7949x; 2.1965x over previous
import functools
import math

import jax
import jax.numpy as jnp
from jax import lax
from jax.experimental import pallas as pl
from jax.experimental.pallas import tpu as pltpu
from jax.experimental.pallas import tpu_sc as plsc

N_OBS = 2097152
OBS_DIM = 3
BINS = 256
COEF = math.sqrt(2.0 * math.log(100000.0))

NC = 2
NS = 16
L = 16
NW = NC * NS
B_PER_W = N_OBS // NW
CH = 8192
N_CH = B_PER_W // CH
G = CH // L


def _ucb_sc_kernel(xh, yh, zh, table_hbm, out_hbm, xv, yv, zv, idx_v,
                   vals_v, out_v, sem):
    wid = lax.axis_index("s") * NC + lax.axis_index("c")
    base = wid * B_PER_W

    def chunk_body(c, _):
        start = base + c * CH
        pltpu.sync_copy(xh.at[pl.ds(start, CH)], xv)
        pltpu.sync_copy(yh.at[pl.ds(start, CH)], yv)
        pltpu.sync_copy(zh.at[pl.ds(start, CH)], zv)

        def group_body(g, _):
            s = pl.ds(g * L, L)
            f0 = (xv[s] * float(BINS)).astype(jnp.int32)
            f1 = (yv[s] * float(BINS)).astype(jnp.int32)
            f2 = (zv[s] * float(BINS)).astype(jnp.int32)
            idx_v[s] = ((f0 << 8) | f1) << 8 | f2
            return 0

        lax.fori_loop(0, G, group_body, 0, unroll=4)

        pltpu.async_copy(table_hbm.at[idx_v], vals_v, sem).wait()

        def bonus_body(g, _):
            s = pl.ds(g * L, L)
            n = vals_v[s]
            i = plsc.bitcast(n, jnp.int32)
            y = plsc.bitcast(jnp.int32(0x5F3759DF) - (i >> 1), jnp.float32)
            hn = 0.5 * n
            y = y * (1.5 - hn * y * y)
            y = y * (1.5 - hn * y * y)
            out_v[s] = COEF * y
            return 0

        lax.fori_loop(0, G, bonus_body, 0, unroll=4)

        pltpu.sync_copy(out_v, out_hbm.at[pl.ds(start, CH)])
        return 0

    lax.fori_loop(0, N_CH, chunk_body, 0)


@jax.jit
def kernel(ob_no, state_counts):
    x = ob_no[:, 0]
    y = ob_no[:, 1]
    z = ob_no[:, 2]
    table_flat = state_counts.reshape(-1)
    mesh = plsc.VectorSubcoreMesh(core_axis_name="c", subcore_axis_name="s",
                                  num_cores=NC, num_subcores=NS)
    run = pl.kernel(
        _ucb_sc_kernel,
        out_type=jax.ShapeDtypeStruct((N_OBS,), jnp.float32),
        mesh=mesh,
        scratch_types=[
            pltpu.VMEM((CH,), jnp.float32),
            pltpu.VMEM((CH,), jnp.float32),
            pltpu.VMEM((CH,), jnp.float32),
            pltpu.VMEM((CH,), jnp.int32),
            pltpu.VMEM((CH,), jnp.float32),
            pltpu.VMEM((CH,), jnp.float32),
            pltpu.SemaphoreType.DMA,
        ],
        compiler_params=pltpu.CompilerParams(needs_layout_passes=False,
                                             use_tc_tiling_on_sc=False),
    )
    return run(x, y, z, table_flat)


# speedup vs baseline: 16.3486x; 1.6691x over previous
import functools
import math

import jax
import jax.numpy as jnp
from jax import lax
from jax.experimental import pallas as pl
from jax.experimental.pallas import tpu as pltpu
from jax.experimental.pallas import tpu_sc as plsc

N_OBS = 2097152
OBS_DIM = 3
BINS = 256
COEF = math.sqrt(2.0 * math.log(100000.0))

NC = 2
NS = 16
L = 16
NW = NC * NS
B_PER_W = N_OBS // NW
CH = 8192
N_CH = B_PER_W // CH
G = CH // L


def _ucb_sc_kernel(xh, yh, zh, table_hbm, out_hbm,
                   xv0, yv0, zv0, xv1, yv1, zv1,
                   idx0, idx1, vals0, vals1, outv0, outv1,
                   semS0, semS1, semG0, semG1, semO0, semO1):
    wid = lax.axis_index("s") * NC + lax.axis_index("c")
    base = wid * B_PER_W

    xv = (xv0, xv1)
    yv = (yv0, yv1)
    zv = (zv0, zv1)
    idx_v = (idx0, idx1)
    vals_v = (vals0, vals1)
    out_v = (outv0, outv1)
    semS = (semS0, semS1)
    semG = (semG0, semG1)
    semO = (semO0, semO1)

    def make_stage(c):
        b = c % 2
        start = base + c * CH
        return [
            pltpu.make_async_copy(xh.at[pl.ds(start, CH)], xv[b], semS[b]),
            pltpu.make_async_copy(yh.at[pl.ds(start, CH)], yv[b], semS[b]),
            pltpu.make_async_copy(zh.at[pl.ds(start, CH)], zv[b], semS[b]),
        ]

    def idx_compute(c):
        b = c % 2

        def group_body(g, _):
            s = pl.ds(g * L, L)
            f0 = (xv[b][s] * float(BINS)).astype(jnp.int32)
            f1 = (yv[b][s] * float(BINS)).astype(jnp.int32)
            f2 = (zv[b][s] * float(BINS)).astype(jnp.int32)
            q = (((f0 << 5) | (f1 >> 3)) << 1) | (f2 >> 7)
            idx_v[b][s] = (((q << 3) | (f1 & 7)) << 7) | (f2 & 127)
            return 0

        lax.fori_loop(0, G, group_body, 0, unroll=4)

    def bonus_compute(c):
        b = c % 2

        def body(g, _):
            s = pl.ds(g * L, L)
            n = vals_v[b][s]
            i = plsc.bitcast(n, jnp.int32)
            y = plsc.bitcast(jnp.int32(0x5F3759DF) - (i >> 1), jnp.float32)
            hn = 0.5 * n
            y = y * (1.5 - hn * y * y)
            y = y * (1.5 - hn * y * y)
            out_v[b][s] = COEF * y
            return 0

        lax.fori_loop(0, G, body, 0, unroll=4)

    stage = {}
    gather = {}
    out_dma = {}

    for c in (0, 1):
        if c < N_CH:
            stage[c] = make_stage(c)
            for h in stage[c]:
                h.start()

    for c in range(N_CH):
        b = c % 2
        for h in stage[c]:
            h.wait()
        idx_compute(c)
        g = pltpu.make_async_copy(table_hbm.at[idx_v[b]], vals_v[b], semG[b])
        g.start()
        gather[c] = g
        if c + 2 < N_CH:
            stage[c + 2] = make_stage(c + 2)
            for h in stage[c + 2]:
                h.start()
        if c >= 1:
            p = c - 1
            if p - 2 >= 0:
                out_dma[p - 2].wait()
            gather[p].wait()
            bonus_compute(p)
            pb = p % 2
            o = pltpu.make_async_copy(
                out_v[pb], out_hbm.at[pl.ds(base + p * CH, CH)], semO[pb])
            o.start()
            out_dma[p] = o

    p = N_CH - 1
    if p - 2 >= 0:
        out_dma[p - 2].wait()
    gather[p].wait()
    bonus_compute(p)
    pb = p % 2
    o = pltpu.make_async_copy(
        out_v[pb], out_hbm.at[pl.ds(base + p * CH, CH)], semO[pb])
    o.start()
    out_dma[p] = o
    for c in (N_CH - 2, N_CH - 1):
        if c >= 0 and c in out_dma and c >= N_CH - 2:
            out_dma[c].wait()


@jax.jit
def kernel(ob_no, state_counts):
    x = ob_no[:, 0]
    y = ob_no[:, 1]
    z = ob_no[:, 2]
    table_flat = (state_counts.reshape(BINS, BINS // 8, 8, 2, 128)
                  .transpose(0, 1, 3, 2, 4).reshape(-1))
    mesh = plsc.VectorSubcoreMesh(core_axis_name="c", subcore_axis_name="s",
                                  num_cores=NC, num_subcores=NS)
    f32 = jnp.float32
    run = pl.kernel(
        _ucb_sc_kernel,
        out_type=jax.ShapeDtypeStruct((N_OBS,), f32),
        mesh=mesh,
        scratch_types=[
            pltpu.VMEM((CH,), f32), pltpu.VMEM((CH,), f32),
            pltpu.VMEM((CH,), f32), pltpu.VMEM((CH,), f32),
            pltpu.VMEM((CH,), f32), pltpu.VMEM((CH,), f32),
            pltpu.VMEM((CH,), jnp.int32), pltpu.VMEM((CH,), jnp.int32),
            pltpu.VMEM((CH,), f32), pltpu.VMEM((CH,), f32),
            pltpu.VMEM((CH,), f32), pltpu.VMEM((CH,), f32),
            pltpu.SemaphoreType.DMA, pltpu.SemaphoreType.DMA,
            pltpu.SemaphoreType.DMA, pltpu.SemaphoreType.DMA,
            pltpu.SemaphoreType.DMA, pltpu.SemaphoreType.DMA,
        ],
        compiler_params=pltpu.CompilerParams(needs_layout_passes=False,
                                             use_tc_tiling_on_sc=False),
    )
    return run(x, y, z, table_flat)
